# baseline (speedup 1.0000x reference)
import functools
import math

import jax
import jax.numpy as jnp
import numpy as np
from jax.experimental import pallas as pl
from jax.sharding import Mesh, PartitionSpec as P

N = 2048
F_IN = 512
HID = 128
OUT = 256
HEADS = 8
ALPHA = 0.2
BM = 512
BP = 256
CK = 512
EPS = 1e-30
LOG2E = math.log2(math.e)
MASK_BIAS = -1e6


def _leaky(x):
    return jnp.maximum(x, ALPHA * x)


def _elu(x):
    return jnp.where(x > 0, x, jnp.exp(x) - 1.0)


def _proj1_kernel(x_ref, w_ref, amat_ref, adj_ref, wh_ref, f_ref, csum_ref,
                  bias_ref):
    i = pl.program_id(0)
    wh = jnp.dot(x_ref[...], w_ref[...], preferred_element_type=jnp.float32)
    wh_ref[...] = wh
    f_ref[...] = jnp.dot(wh, amat_ref[...], preferred_element_type=jnp.float32)
    bias_ref[...] = jnp.where(adj_ref[...] > 0, 0.0, MASK_BIAS)
    part = jnp.sum(wh, axis=0, keepdims=True)

    @pl.when(i == 0)
    def _():
        csum_ref[...] = part

    @pl.when(i > 0)
    def _():
        csum_ref[...] += part


def _attn1_kernel(wh_ref, bias_ref, f1_ref, f2t_ref, wout_ref, aomat_ref,
                  csum_ref, wh2_ref, fo_ref, csum2_ref):
    i = pl.program_id(0)
    h = pl.program_id(1)
    f1 = f1_ref[pl.ds(i * BM, BM), :]
    onehot = (jax.lax.broadcasted_iota(jnp.int32, (HEADS, 1), 0) == h
              ).astype(jnp.float32)
    f1c = jnp.dot(f1, onehot, preferred_element_type=jnp.float32)

    num = jnp.zeros((BM, HID), jnp.float32)
    s = jnp.zeros((BM, 1), jnp.float32)
    for c in range(N // CK):
        f2c = f2t_ref[pl.ds(h, 1), c * CK:(c + 1) * CK]
        biasc = bias_ref[:, c * CK:(c + 1) * CK]
        pc = jnp.exp2(_leaky(f1c + f2c) + biasc)
        s = s + jnp.sum(pc, axis=1, keepdims=True)
        whc = wh_ref[pl.ds(c * CK, CK), pl.ds(h * HID, HID)]
        num = num + jnp.dot(pc, whc, preferred_element_type=jnp.float32)

    s = s + (N * EPS)
    csum_h = csum_ref[:, pl.ds(h * HID, HID)]
    hh = _elu((num + EPS * csum_h) * (1.0 / s))
    wout_h = wout_ref[pl.ds(h * HID, HID), :]
    contrib = jnp.dot(hh, wout_h, preferred_element_type=jnp.float32)

    @pl.when(h == 0)
    def _():
        wh2_ref[...] = contrib

    @pl.when(h > 0)
    def _():
        wh2_ref[...] += contrib

    @pl.when(h == HEADS - 1)
    def _():
        wh2 = wh2_ref[...]
        fo_ref[...] = jnp.dot(wh2, aomat_ref[...],
                              preferred_element_type=jnp.float32)
        part2 = jnp.sum(wh2, axis=0, keepdims=True)

        @pl.when(i == 0)
        def _():
            csum2_ref[...] = part2

        @pl.when(i > 0)
        def _():
            csum2_ref[...] += part2


def _attn2_kernel(wh2_ref, bias_ref, fo_ref, fot_ref, csum2_ref, out_ref):
    i = pl.program_id(0)
    f1 = fo_ref[pl.ds(i * BM, BM), 0:1]

    num = jnp.zeros((BM, OUT), jnp.float32)
    s = jnp.zeros((BM, 1), jnp.float32)
    for c in range(N // CK):
        f2c = fot_ref[1:2, c * CK:(c + 1) * CK]
        biasc = bias_ref[:, c * CK:(c + 1) * CK]
        pc = jnp.exp2(_leaky(f1 + f2c) + biasc)
        s = s + jnp.sum(pc, axis=1, keepdims=True)
        wh2c = wh2_ref[c * CK:(c + 1) * CK, :]
        num = num + jnp.dot(pc, wh2c, preferred_element_type=jnp.float32)

    s = s + (N * EPS)
    out_ref[...] = _elu((num + EPS * csum2_ref[...]) * (1.0 / s))


def _gat_local(x_l, adj_l, w_all, amat, wout, aomat):
    f32 = jnp.float32
    ln = x_l.shape[0]

    wh_l, f_l, csum_l, bias_l = pl.pallas_call(
        _proj1_kernel,
        grid=(ln // BP,),
        in_specs=[
            pl.BlockSpec((BP, F_IN), lambda i: (i, 0)),
            pl.BlockSpec((F_IN, HEADS * HID), lambda i: (0, 0)),
            pl.BlockSpec((HEADS * HID, 2 * HEADS), lambda i: (0, 0)),
            pl.BlockSpec((BP, N), lambda i: (i, 0)),
        ],
        out_specs=[
            pl.BlockSpec((BP, HEADS * HID), lambda i: (i, 0)),
            pl.BlockSpec((BP, 2 * HEADS), lambda i: (i, 0)),
            pl.BlockSpec((1, HEADS * HID), lambda i: (0, 0)),
            pl.BlockSpec((BP, N), lambda i: (i, 0)),
        ],
        out_shape=[
            jax.ShapeDtypeStruct((ln, HEADS * HID), f32),
            jax.ShapeDtypeStruct((ln, 2 * HEADS), f32),
            jax.ShapeDtypeStruct((1, HEADS * HID), f32),
            jax.ShapeDtypeStruct((ln, N), f32),
        ],
    )(x_l, w_all, amat, adj_l)

    wh = jax.lax.all_gather(wh_l, 'd', axis=0, tiled=True)
    f_full = jax.lax.all_gather(f_l, 'd', axis=0, tiled=True)
    csum = jax.lax.psum(csum_l, 'd')

    f1 = f_l[:, :HEADS]
    f2t = f_full[:, HEADS:].T

    wh2_l, fo_l, csum2_l = pl.pallas_call(
        _attn1_kernel,
        grid=(ln // BM, HEADS),
        in_specs=[
            pl.BlockSpec((N, HEADS * HID), lambda i, h: (0, 0)),
            pl.BlockSpec((BM, N), lambda i, h: (i, 0)),
            pl.BlockSpec((ln, HEADS), lambda i, h: (0, 0)),
            pl.BlockSpec((HEADS, N), lambda i, h: (0, 0)),
            pl.BlockSpec((HEADS * HID, OUT), lambda i, h: (0, 0)),
            pl.BlockSpec((OUT, 2), lambda i, h: (0, 0)),
            pl.BlockSpec((1, HEADS * HID), lambda i, h: (0, 0)),
        ],
        out_specs=[
            pl.BlockSpec((BM, OUT), lambda i, h: (i, 0)),
            pl.BlockSpec((BM, 2), lambda i, h: (i, 0)),
            pl.BlockSpec((1, OUT), lambda i, h: (0, 0)),
        ],
        out_shape=[
            jax.ShapeDtypeStruct((ln, OUT), f32),
            jax.ShapeDtypeStruct((ln, 2), f32),
            jax.ShapeDtypeStruct((1, OUT), f32),
        ],
    )(wh, bias_l, f1, f2t, wout, aomat, csum)

    wh2 = jax.lax.all_gather(wh2_l, 'd', axis=0, tiled=True)
    fo_full = jax.lax.all_gather(fo_l, 'd', axis=0, tiled=True)
    csum2 = jax.lax.psum(csum2_l, 'd')
    fot = fo_full.T

    out_l = pl.pallas_call(
        _attn2_kernel,
        grid=(ln // BM,),
        in_specs=[
            pl.BlockSpec((N, OUT), lambda i: (0, 0)),
            pl.BlockSpec((BM, N), lambda i: (i, 0)),
            pl.BlockSpec((ln, 2), lambda i: (0, 0)),
            pl.BlockSpec((2, N), lambda i: (0, 0)),
            pl.BlockSpec((1, OUT), lambda i: (0, 0)),
        ],
        out_specs=pl.BlockSpec((BM, OUT), lambda i: (i, 0)),
        out_shape=jax.ShapeDtypeStruct((ln, OUT), f32),
    )(wh2, bias_l, fo_l, fot, csum2)

    return out_l


def kernel(x, adj, W_heads, a_heads, W_out, a_out):
    f32 = jnp.float32
    w_all = jnp.transpose(W_heads, (1, 0, 2)).reshape(F_IN, HEADS * HID)
    a1 = a_heads[:, :HID, 0]
    a2 = a_heads[:, HID:, 0]
    eye = jnp.eye(HEADS, dtype=f32)
    amat1 = (a1[:, :, None] * eye[:, None, :]).reshape(HEADS * HID, HEADS)
    amat2 = (a2[:, :, None] * eye[:, None, :]).reshape(HEADS * HID, HEADS)
    amat = jnp.concatenate([amat1, amat2], axis=1) * LOG2E
    aomat = jnp.concatenate([a_out[:OUT], a_out[OUT:]], axis=1) * LOG2E

    devs = jax.devices()
    ndev = 2 if len(devs) >= 2 else 1
    mesh = Mesh(np.array(devs[:ndev]), ('d',))
    fn = jax.shard_map(
        _gat_local,
        mesh=mesh,
        in_specs=(P('d', None), P('d', None), P(None, None), P(None, None),
                  P(None, None), P(None, None)),
        out_specs=P('d', None),
        check_vma=False,
    )
    return fn(x, adj, w_all, amat, W_out, aomat)


# speedup vs baseline: 6.3824x; 6.3824x over previous
import math

import jax
import jax.numpy as jnp
from jax.experimental import pallas as pl

N = 2048
F_IN = 512
HID = 128
OUT = 256
HEADS = 8
ALPHA = 0.2
BM = 512
BP = 256
CK = 512
EPS = 1e-30
LOG2E = math.log2(math.e)
MASK_BIAS = -1e6


def _leaky(x):
    return jnp.maximum(x, ALPHA * x)


def _elu(x):
    return jnp.where(x > 0, x, jnp.exp(x) - 1.0)


def _proj1_kernel(x_ref, w_ref, amat_ref, adj_ref, wh_ref, f_ref, csum_ref,
                  bias_ref):
    i = pl.program_id(0)
    wh = jnp.dot(x_ref[...], w_ref[...], preferred_element_type=jnp.float32)
    wh_ref[...] = wh
    f_ref[...] = jnp.dot(wh, amat_ref[...], preferred_element_type=jnp.float32)
    bias_ref[...] = jnp.where(adj_ref[...] > 0, 0.0, MASK_BIAS)
    part = jnp.sum(wh, axis=0, keepdims=True)

    @pl.when(i == 0)
    def _():
        csum_ref[...] = part

    @pl.when(i > 0)
    def _():
        csum_ref[...] += part


def _attn1_kernel(wh_ref, bias_ref, f1_ref, f2t_ref, wout_ref, aomat_ref,
                  csum_ref, wh2_ref, fo_ref, csum2_ref):
    i = pl.program_id(0)
    hhs = []
    for h in range(HEADS):
        f1c = f1_ref[:, h:h + 1]
        num = jnp.zeros((BM, HID), jnp.float32)
        s = jnp.zeros((BM, 1), jnp.float32)
        for c in range(N // CK):
            f2c = f2t_ref[h:h + 1, c * CK:(c + 1) * CK]
            biasc = bias_ref[:, c * CK:(c + 1) * CK]
            pc = jnp.exp2(_leaky(f1c + f2c) + biasc)
            s = s + jnp.sum(pc, axis=1, keepdims=True)
            whc = wh_ref[c * CK:(c + 1) * CK, h * HID:(h + 1) * HID]
            num = num + jnp.dot(pc, whc,
                                preferred_element_type=jnp.float32)
        s = s + (N * EPS)
        csum_h = csum_ref[:, h * HID:(h + 1) * HID]
        hhs.append(_elu((num + EPS * csum_h) * (1.0 / s)))

    hcat = jnp.concatenate(hhs, axis=1)
    wh2 = jnp.dot(hcat, wout_ref[...], preferred_element_type=jnp.float32)
    wh2_ref[...] = wh2
    fo_ref[...] = jnp.dot(wh2, aomat_ref[...],
                          preferred_element_type=jnp.float32)
    part2 = jnp.sum(wh2, axis=0, keepdims=True)

    @pl.when(i == 0)
    def _():
        csum2_ref[...] = part2

    @pl.when(i > 0)
    def _():
        csum2_ref[...] += part2


def _attn2_kernel(wh2_ref, bias_ref, fo_ref, fot_ref, csum2_ref, out_ref):
    i = pl.program_id(0)
    f1 = fo_ref[pl.ds(i * BM, BM), 0:1]

    num = jnp.zeros((BM, OUT), jnp.float32)
    s = jnp.zeros((BM, 1), jnp.float32)
    for c in range(N // CK):
        f2c = fot_ref[1:2, c * CK:(c + 1) * CK]
        biasc = bias_ref[:, c * CK:(c + 1) * CK]
        pc = jnp.exp2(_leaky(f1 + f2c) + biasc)
        s = s + jnp.sum(pc, axis=1, keepdims=True)
        wh2c = wh2_ref[c * CK:(c + 1) * CK, :]
        num = num + jnp.dot(pc, wh2c, preferred_element_type=jnp.float32)

    s = s + (N * EPS)
    out_ref[...] = _elu((num + EPS * csum2_ref[...]) * (1.0 / s))


def kernel(x, adj, W_heads, a_heads, W_out, a_out):
    f32 = jnp.float32
    w_all = jnp.transpose(W_heads, (1, 0, 2)).reshape(F_IN, HEADS * HID)
    a1 = a_heads[:, :HID, 0]
    a2 = a_heads[:, HID:, 0]
    eye = jnp.eye(HEADS, dtype=f32)
    amat1 = (a1[:, :, None] * eye[:, None, :]).reshape(HEADS * HID, HEADS)
    amat2 = (a2[:, :, None] * eye[:, None, :]).reshape(HEADS * HID, HEADS)
    amat = jnp.concatenate([amat1, amat2], axis=1) * LOG2E
    aomat = jnp.concatenate([a_out[:OUT], a_out[OUT:]], axis=1) * LOG2E

    wh, f, csum, bias = pl.pallas_call(
        _proj1_kernel,
        grid=(N // BP,),
        in_specs=[
            pl.BlockSpec((BP, F_IN), lambda i: (i, 0)),
            pl.BlockSpec((F_IN, HEADS * HID), lambda i: (0, 0)),
            pl.BlockSpec((HEADS * HID, 2 * HEADS), lambda i: (0, 0)),
            pl.BlockSpec((BP, N), lambda i: (i, 0)),
        ],
        out_specs=[
            pl.BlockSpec((BP, HEADS * HID), lambda i: (i, 0)),
            pl.BlockSpec((BP, 2 * HEADS), lambda i: (i, 0)),
            pl.BlockSpec((1, HEADS * HID), lambda i: (0, 0)),
            pl.BlockSpec((BP, N), lambda i: (i, 0)),
        ],
        out_shape=[
            jax.ShapeDtypeStruct((N, HEADS * HID), f32),
            jax.ShapeDtypeStruct((N, 2 * HEADS), f32),
            jax.ShapeDtypeStruct((1, HEADS * HID), f32),
            jax.ShapeDtypeStruct((N, N), f32),
        ],
    )(x, w_all, amat, adj)

    f1 = f[:, :HEADS]
    f2t = f[:, HEADS:].T

    wh2, fo, csum2 = pl.pallas_call(
        _attn1_kernel,
        grid=(N // BM,),
        in_specs=[
            pl.BlockSpec((N, HEADS * HID), lambda i: (0, 0)),
            pl.BlockSpec((BM, N), lambda i: (i, 0)),
            pl.BlockSpec((BM, HEADS), lambda i: (i, 0)),
            pl.BlockSpec((HEADS, N), lambda i: (0, 0)),
            pl.BlockSpec((HEADS * HID, OUT), lambda i: (0, 0)),
            pl.BlockSpec((OUT, 2), lambda i: (0, 0)),
            pl.BlockSpec((1, HEADS * HID), lambda i: (0, 0)),
        ],
        out_specs=[
            pl.BlockSpec((BM, OUT), lambda i: (i, 0)),
            pl.BlockSpec((BM, 2), lambda i: (i, 0)),
            pl.BlockSpec((1, OUT), lambda i: (0, 0)),
        ],
        out_shape=[
            jax.ShapeDtypeStruct((N, OUT), f32),
            jax.ShapeDtypeStruct((N, 2), f32),
            jax.ShapeDtypeStruct((1, OUT), f32),
        ],
    )(wh, bias, f1, f2t, W_out, aomat, csum)

    fot = fo.T

    out = pl.pallas_call(
        _attn2_kernel,
        grid=(N // BM,),
        in_specs=[
            pl.BlockSpec((N, OUT), lambda i: (0, 0)),
            pl.BlockSpec((BM, N), lambda i: (i, 0)),
            pl.BlockSpec((N, 2), lambda i: (0, 0)),
            pl.BlockSpec((2, N), lambda i: (0, 0)),
            pl.BlockSpec((1, OUT), lambda i: (0, 0)),
        ],
        out_specs=pl.BlockSpec((BM, OUT), lambda i: (i, 0)),
        out_shape=jax.ShapeDtypeStruct((N, OUT), f32),
    )(wh2, bias, fo, fot, csum2)

    return out


# speedup vs baseline: 6.6233x; 1.0377x over previous
import math

import jax
import jax.numpy as jnp
from jax.experimental import pallas as pl

N = 2048
F_IN = 512
HID = 128
OUT = 256
HEADS = 8
ALPHA = 0.2
BM = 512
BP = 256
CK = 512
EPS = 1e-30
LOG2E = math.log2(math.e)
MASK_BIAS = -1e6


def _leaky(x):
    return jnp.maximum(x, ALPHA * x)


def _elu(x):
    return jnp.where(x > 0, x, jnp.exp(x) - 1.0)


def _proj1_kernel(x_ref, w_ref, amat_ref, adj_ref, wh_ref, f1_ref, f2t_ref,
                  csum_ref, bias_ref):
    i = pl.program_id(0)
    wh = jnp.dot(x_ref[...], w_ref[...], preferred_element_type=jnp.float32)
    wh_ref[...] = wh
    f = jnp.dot(wh, amat_ref[...], preferred_element_type=jnp.float32)
    f1_ref[...] = f[:, :HEADS]
    f2t_ref[...] = f[:, HEADS:].T
    bias_ref[...] = jnp.where(adj_ref[...] > 0, 0.0, MASK_BIAS)
    part = jnp.sum(wh, axis=0, keepdims=True)

    @pl.when(i == 0)
    def _():
        csum_ref[...] = part

    @pl.when(i > 0)
    def _():
        csum_ref[...] += part


def _attn1_kernel(wh_ref, bias_ref, f1_ref, f2t_ref, wout_ref, aomat_ref,
                  csum_ref, wh2_ref, fo_ref, fot_ref, csum2_ref):
    i = pl.program_id(0)
    hhs = []
    for h in range(HEADS):
        f1c = f1_ref[:, h:h + 1]
        num = jnp.zeros((BM, HID), jnp.float32)
        s = jnp.zeros((BM, 1), jnp.float32)
        for c in range(N // CK):
            f2c = f2t_ref[h:h + 1, c * CK:(c + 1) * CK]
            biasc = bias_ref[:, c * CK:(c + 1) * CK]
            pc = jnp.exp2(_leaky(f1c + f2c) + biasc)
            s = s + jnp.sum(pc, axis=1, keepdims=True)
            whc = wh_ref[c * CK:(c + 1) * CK, h * HID:(h + 1) * HID]
            num = num + jnp.dot(pc, whc,
                                preferred_element_type=jnp.float32)
        s = s + (N * EPS)
        csum_h = csum_ref[:, h * HID:(h + 1) * HID]
        hhs.append(_elu((num + EPS * csum_h) * (1.0 / s)))

    hcat = jnp.concatenate(hhs, axis=1)
    wh2 = jnp.dot(hcat, wout_ref[...], preferred_element_type=jnp.float32)
    wh2_ref[...] = wh2
    fo = jnp.dot(wh2, aomat_ref[...],
                 preferred_element_type=jnp.float32)
    fo_ref[...] = fo
    fot_ref[...] = fo.T
    part2 = jnp.sum(wh2, axis=0, keepdims=True)

    @pl.when(i == 0)
    def _():
        csum2_ref[...] = part2

    @pl.when(i > 0)
    def _():
        csum2_ref[...] += part2


def _attn2_kernel(wh2_ref, bias_ref, fo_ref, fot_ref, csum2_ref, out_ref):
    f1 = fo_ref[:, 0:1]

    num = jnp.zeros((BM, OUT), jnp.float32)
    s = jnp.zeros((BM, 1), jnp.float32)
    for c in range(N // CK):
        f2c = fot_ref[1:2, c * CK:(c + 1) * CK]
        biasc = bias_ref[:, c * CK:(c + 1) * CK]
        pc = jnp.exp2(_leaky(f1 + f2c) + biasc)
        s = s + jnp.sum(pc, axis=1, keepdims=True)
        wh2c = wh2_ref[c * CK:(c + 1) * CK, :]
        num = num + jnp.dot(pc, wh2c, preferred_element_type=jnp.float32)

    s = s + (N * EPS)
    out_ref[...] = _elu((num + EPS * csum2_ref[...]) * (1.0 / s))


def kernel(x, adj, W_heads, a_heads, W_out, a_out):
    f32 = jnp.float32
    w_all = jnp.transpose(W_heads, (1, 0, 2)).reshape(F_IN, HEADS * HID)
    a1 = a_heads[:, :HID, 0]
    a2 = a_heads[:, HID:, 0]
    eye = jnp.eye(HEADS, dtype=f32)
    amat1 = (a1[:, :, None] * eye[:, None, :]).reshape(HEADS * HID, HEADS)
    amat2 = (a2[:, :, None] * eye[:, None, :]).reshape(HEADS * HID, HEADS)
    amat = jnp.concatenate([amat1, amat2], axis=1) * LOG2E
    aomat = jnp.concatenate([a_out[:OUT], a_out[OUT:]], axis=1) * LOG2E

    wh, f1, f2t, csum, bias = pl.pallas_call(
        _proj1_kernel,
        grid=(N // BP,),
        in_specs=[
            pl.BlockSpec((BP, F_IN), lambda i: (i, 0)),
            pl.BlockSpec((F_IN, HEADS * HID), lambda i: (0, 0)),
            pl.BlockSpec((HEADS * HID, 2 * HEADS), lambda i: (0, 0)),
            pl.BlockSpec((BP, N), lambda i: (i, 0)),
        ],
        out_specs=[
            pl.BlockSpec((BP, HEADS * HID), lambda i: (i, 0)),
            pl.BlockSpec((BP, HEADS), lambda i: (i, 0)),
            pl.BlockSpec((HEADS, BP), lambda i: (0, i)),
            pl.BlockSpec((1, HEADS * HID), lambda i: (0, 0)),
            pl.BlockSpec((BP, N), lambda i: (i, 0)),
        ],
        out_shape=[
            jax.ShapeDtypeStruct((N, HEADS * HID), f32),
            jax.ShapeDtypeStruct((N, HEADS), f32),
            jax.ShapeDtypeStruct((HEADS, N), f32),
            jax.ShapeDtypeStruct((1, HEADS * HID), f32),
            jax.ShapeDtypeStruct((N, N), f32),
        ],
    )(x, w_all, amat, adj)

    wh2, fo, fot, csum2 = pl.pallas_call(
        _attn1_kernel,
        grid=(N // BM,),
        in_specs=[
            pl.BlockSpec((N, HEADS * HID), lambda i: (0, 0)),
            pl.BlockSpec((BM, N), lambda i: (i, 0)),
            pl.BlockSpec((BM, HEADS), lambda i: (i, 0)),
            pl.BlockSpec((HEADS, N), lambda i: (0, 0)),
            pl.BlockSpec((HEADS * HID, OUT), lambda i: (0, 0)),
            pl.BlockSpec((OUT, 2), lambda i: (0, 0)),
            pl.BlockSpec((1, HEADS * HID), lambda i: (0, 0)),
        ],
        out_specs=[
            pl.BlockSpec((BM, OUT), lambda i: (i, 0)),
            pl.BlockSpec((BM, 2), lambda i: (i, 0)),
            pl.BlockSpec((2, BM), lambda i: (0, i)),
            pl.BlockSpec((1, OUT), lambda i: (0, 0)),
        ],
        out_shape=[
            jax.ShapeDtypeStruct((N, OUT), f32),
            jax.ShapeDtypeStruct((N, 2), f32),
            jax.ShapeDtypeStruct((2, N), f32),
            jax.ShapeDtypeStruct((1, OUT), f32),
        ],
    )(wh, bias, f1, f2t, W_out, aomat, csum)

    out = pl.pallas_call(
        _attn2_kernel,
        grid=(N // BM,),
        in_specs=[
            pl.BlockSpec((N, OUT), lambda i: (0, 0)),
            pl.BlockSpec((BM, N), lambda i: (i, 0)),
            pl.BlockSpec((BM, 2), lambda i: (i, 0)),
            pl.BlockSpec((2, N), lambda i: (0, 0)),
            pl.BlockSpec((1, OUT), lambda i: (0, 0)),
        ],
        out_specs=pl.BlockSpec((BM, OUT), lambda i: (i, 0)),
        out_shape=jax.ShapeDtypeStruct((N, OUT), f32),
    )(wh2, bias, fo, fot, csum2)

    return out


# speedup vs baseline: 7.5148x; 1.1346x over previous
import math

import jax
import jax.numpy as jnp
from jax.experimental import pallas as pl

N = 2048
F_IN = 512
HID = 128
OUT = 256
HEADS = 8
ALPHA = 0.2
BM = 512
BP = 256
CK = 512
AUG = 2 * HID
EPS = 1e-30
LOG2E = math.log2(math.e)
MASK_BIAS = -1e6


def _leaky(x):
    return jnp.maximum(x, ALPHA * x)


def _elu(x):
    return jnp.where(x > 0, x, jnp.exp(x) - 1.0)


def _proj1_kernel(x_ref, w_ref, amat_ref, adj_ref, wh_ref, f1_ref, f2t_ref,
                  csum_ref, bias_ref):
    i = pl.program_id(0)
    wh = jnp.dot(x_ref[...], w_ref[...], preferred_element_type=jnp.float32)
    ones = jnp.ones((wh.shape[0], HID), jnp.float32)
    pieces = []
    for h in range(HEADS):
        pieces.append(wh[:, h * HID:(h + 1) * HID])
        pieces.append(ones)
    whaug = jnp.concatenate(pieces, axis=1)
    wh_ref[...] = whaug
    f = jnp.dot(wh, amat_ref[...], preferred_element_type=jnp.float32)
    f1_ref[...] = f[:, :HEADS]
    f2t_ref[...] = f[:, HEADS:].T
    bias_ref[...] = jnp.where(adj_ref[...] > 0, 0.0, MASK_BIAS)
    part = jnp.sum(whaug, axis=0, keepdims=True)

    @pl.when(i == 0)
    def _():
        csum_ref[...] = part

    @pl.when(i > 0)
    def _():
        csum_ref[...] += part


def _attn1_kernel(wh_ref, bias_ref, f1_ref, f2t_ref, wout_ref, aomat_ref,
                  csum_ref, wh2_ref, fo_ref, fot_ref, csum2_ref):
    i = pl.program_id(0)
    hhs = []
    for h in range(HEADS):
        f1c = f1_ref[:, h:h + 1]
        numaug = jnp.zeros((BM, AUG), jnp.float32)
        for c in range(N // CK):
            f2c = f2t_ref[h:h + 1, c * CK:(c + 1) * CK]
            biasc = bias_ref[:, c * CK:(c + 1) * CK]
            pc = jnp.exp2(_leaky(f1c + f2c) + biasc)
            whc = wh_ref[c * CK:(c + 1) * CK, h * AUG:(h + 1) * AUG]
            numaug = numaug + jnp.dot(pc, whc,
                                      preferred_element_type=jnp.float32)
        tot = numaug + EPS * csum_ref[:, h * AUG:(h + 1) * AUG]
        num = tot[:, :HID]
        s = tot[:, HID:HID + 1]
        hhs.append(_elu(num * (1.0 / s)))

    hcat = jnp.concatenate(hhs, axis=1)
    wh2 = jnp.dot(hcat, wout_ref[...], preferred_element_type=jnp.float32)
    wh2_ref[...] = wh2
    fo = jnp.dot(wh2, aomat_ref[...],
                 preferred_element_type=jnp.float32)
    fo_ref[...] = fo
    fot_ref[...] = fo.T
    part2 = jnp.sum(wh2, axis=0, keepdims=True)

    @pl.when(i == 0)
    def _():
        csum2_ref[...] = part2

    @pl.when(i > 0)
    def _():
        csum2_ref[...] += part2


def _attn2_kernel(wh2_ref, bias_ref, fo_ref, fot_ref, csum2_ref, out_ref):
    f1 = fo_ref[:, 0:1]

    num = jnp.zeros((BM, OUT), jnp.float32)
    s = jnp.zeros((BM, 1), jnp.float32)
    for c in range(N // CK):
        f2c = fot_ref[1:2, c * CK:(c + 1) * CK]
        biasc = bias_ref[:, c * CK:(c + 1) * CK]
        pc = jnp.exp2(_leaky(f1 + f2c) + biasc)
        s = s + jnp.sum(pc, axis=1, keepdims=True)
        wh2c = wh2_ref[c * CK:(c + 1) * CK, :]
        num = num + jnp.dot(pc, wh2c, preferred_element_type=jnp.float32)

    s = s + (N * EPS)
    out_ref[...] = _elu((num + EPS * csum2_ref[...]) * (1.0 / s))


def kernel(x, adj, W_heads, a_heads, W_out, a_out):
    f32 = jnp.float32
    w_all = jnp.transpose(W_heads, (1, 0, 2)).reshape(F_IN, HEADS * HID)
    a1 = a_heads[:, :HID, 0]
    a2 = a_heads[:, HID:, 0]
    eye = jnp.eye(HEADS, dtype=f32)
    amat1 = (a1[:, :, None] * eye[:, None, :]).reshape(HEADS * HID, HEADS)
    amat2 = (a2[:, :, None] * eye[:, None, :]).reshape(HEADS * HID, HEADS)
    amat = jnp.concatenate([amat1, amat2], axis=1) * LOG2E
    aomat = jnp.concatenate([a_out[:OUT], a_out[OUT:]], axis=1) * LOG2E

    wh, f1, f2t, csum, bias = pl.pallas_call(
        _proj1_kernel,
        grid=(N // BP,),
        in_specs=[
            pl.BlockSpec((BP, F_IN), lambda i: (i, 0)),
            pl.BlockSpec((F_IN, HEADS * HID), lambda i: (0, 0)),
            pl.BlockSpec((HEADS * HID, 2 * HEADS), lambda i: (0, 0)),
            pl.BlockSpec((BP, N), lambda i: (i, 0)),
        ],
        out_specs=[
            pl.BlockSpec((BP, HEADS * AUG), lambda i: (i, 0)),
            pl.BlockSpec((BP, HEADS), lambda i: (i, 0)),
            pl.BlockSpec((HEADS, BP), lambda i: (0, i)),
            pl.BlockSpec((1, HEADS * AUG), lambda i: (0, 0)),
            pl.BlockSpec((BP, N), lambda i: (i, 0)),
        ],
        out_shape=[
            jax.ShapeDtypeStruct((N, HEADS * AUG), f32),
            jax.ShapeDtypeStruct((N, HEADS), f32),
            jax.ShapeDtypeStruct((HEADS, N), f32),
            jax.ShapeDtypeStruct((1, HEADS * AUG), f32),
            jax.ShapeDtypeStruct((N, N), f32),
        ],
    )(x, w_all, amat, adj)

    wh2, fo, fot, csum2 = pl.pallas_call(
        _attn1_kernel,
        grid=(N // BM,),
        in_specs=[
            pl.BlockSpec((N, HEADS * AUG), lambda i: (0, 0)),
            pl.BlockSpec((BM, N), lambda i: (i, 0)),
            pl.BlockSpec((BM, HEADS), lambda i: (i, 0)),
            pl.BlockSpec((HEADS, N), lambda i: (0, 0)),
            pl.BlockSpec((HEADS * HID, OUT), lambda i: (0, 0)),
            pl.BlockSpec((OUT, 2), lambda i: (0, 0)),
            pl.BlockSpec((1, HEADS * AUG), lambda i: (0, 0)),
        ],
        out_specs=[
            pl.BlockSpec((BM, OUT), lambda i: (i, 0)),
            pl.BlockSpec((BM, 2), lambda i: (i, 0)),
            pl.BlockSpec((2, BM), lambda i: (0, i)),
            pl.BlockSpec((1, OUT), lambda i: (0, 0)),
        ],
        out_shape=[
            jax.ShapeDtypeStruct((N, OUT), f32),
            jax.ShapeDtypeStruct((N, 2), f32),
            jax.ShapeDtypeStruct((2, N), f32),
            jax.ShapeDtypeStruct((1, OUT), f32),
        ],
    )(wh, bias, f1, f2t, W_out, aomat, csum)

    out = pl.pallas_call(
        _attn2_kernel,
        grid=(N // BM,),
        in_specs=[
            pl.BlockSpec((N, OUT), lambda i: (0, 0)),
            pl.BlockSpec((BM, N), lambda i: (i, 0)),
            pl.BlockSpec((BM, 2), lambda i: (i, 0)),
            pl.BlockSpec((2, N), lambda i: (0, 0)),
            pl.BlockSpec((1, OUT), lambda i: (0, 0)),
        ],
        out_specs=pl.BlockSpec((BM, OUT), lambda i: (i, 0)),
        out_shape=jax.ShapeDtypeStruct((N, OUT), f32),
    )(wh2, bias, fo, fot, csum2)

    return out
